# Initial kernel scaffold; baseline (speedup 1.0000x reference)
import jax
import jax.numpy as jnp
from jax.experimental import pallas as pl


def kernel(x, embedding):
    raise NotImplementedError("write your pallas kernel here")


# speedup vs baseline: 1.5166x; 1.5166x over previous
import functools

import jax
import jax.numpy as jnp
from jax import lax
from jax.experimental import pallas as pl
from jax.experimental.pallas import tpu as pltpu
from jax.experimental.pallas import tpu_sc as plsc

D_MODEL = 512
_NC = 2
_NS = 16
_NW = _NC * _NS
_B = 1024 * 200
_BPW = _B // _NW
_C = 64
_G = _BPW // _C

_mesh = plsc.VectorSubcoreMesh(core_axis_name="c", subcore_axis_name="s")


@functools.partial(
    pl.kernel,
    out_type=jax.ShapeDtypeStruct((_NW, _G, _C, D_MODEL), jnp.float32),
    mesh=_mesh,
    scratch_types=[
        pltpu.VMEM((_G, _C), jnp.int32),
        pltpu.VMEM((_C, D_MODEL), jnp.float32),
        pltpu.SemaphoreType.DMA,
    ],
)
def _emb_lookup(table_hbm, idx_hbm, out_hbm, idx_v, rows_v, sem):
    wid = lax.axis_index("s") * _NC + lax.axis_index("c")
    pltpu.sync_copy(idx_hbm.at[wid], idx_v)

    def body(g):
        pltpu.async_copy(table_hbm.at[idx_v.at[g]], rows_v, sem).wait()
        pltpu.sync_copy(rows_v, out_hbm.at[wid, g])

    pl.loop(0, _G)(body)


def kernel(x, embedding):
    idx = x.astype(jnp.int32).reshape(_NW, _G, _C)
    out = _emb_lookup(embedding, idx)
    return out.reshape(x.shape + (D_MODEL,))


# speedup vs baseline: 1.8072x; 1.1916x over previous
import functools

import jax
import jax.numpy as jnp
from jax import lax
from jax.experimental import pallas as pl
from jax.experimental.pallas import tpu as pltpu
from jax.experimental.pallas import tpu_sc as plsc

D_MODEL = 512
_NC = 2
_NS = 16
_NW = _NC * _NS
_B = 1024 * 200
_BPW = _B // _NW
_C = 40
_G = _BPW // _C
_NBUF = 4

_mesh = plsc.VectorSubcoreMesh(core_axis_name="c", subcore_axis_name="s")


@functools.partial(
    pl.kernel,
    out_type=jax.ShapeDtypeStruct((_NW, _G, _C, D_MODEL), jnp.float32),
    mesh=_mesh,
    scratch_types=[
        pltpu.VMEM((_G, _C), jnp.int32),
        [pltpu.VMEM((_C, D_MODEL), jnp.float32) for _ in range(_NBUF)],
        [pltpu.SemaphoreType.DMA for _ in range(_NBUF)],
        [pltpu.SemaphoreType.DMA for _ in range(_NBUF)],
    ],
)
def _emb_lookup(table_hbm, idx_hbm, out_hbm, idx_v, rows, gsem, ssem):
    wid = lax.axis_index("s") * _NC + lax.axis_index("c")
    pltpu.sync_copy(idx_hbm.at[wid], idx_v)

    def gstart(b, c):
        pltpu.async_copy(table_hbm.at[idx_v.at[c]], rows[b], gsem[b])

    def gwait(b, c):
        pltpu.make_async_copy(table_hbm.at[idx_v.at[c]], rows[b], gsem[b]).wait()

    def sstart(b, c):
        pltpu.async_copy(rows[b], out_hbm.at[wid, c], ssem[b])

    def swait(b, c):
        pltpu.make_async_copy(rows[b], out_hbm.at[wid, c], ssem[b]).wait()

    for b in range(_NBUF):
        gstart(b, b)

    def steady(g):
        for b in range(_NBUF):
            gwait(b, g + b)
            sstart(b, g + b)
        for b in range(_NBUF):
            swait(b, g + b)
            gstart(b, g + b + _NBUF)

    pl.loop(0, _G - _NBUF, step=_NBUF)(steady)

    for b in range(_NBUF):
        gwait(b, _G - _NBUF + b)
        sstart(b, _G - _NBUF + b)
    for b in range(_NBUF):
        swait(b, _G - _NBUF + b)


def kernel(x, embedding):
    idx = x.astype(jnp.int32).reshape(_NW, _G, _C)
    out = _emb_lookup(embedding, idx)
    return out.reshape(x.shape + (D_MODEL,))
